# Initial kernel scaffold; baseline (speedup 1.0000x reference)
import jax
import jax.numpy as jnp
from jax.experimental import pallas as pl


def kernel(hidden_states, embedding, output_tokens, presence_penalties, frequency_penalties, repetition_penalties, temperatures, top_ps, top_ks, top_as):
    raise NotImplementedError("write your pallas kernel here")


# speedup vs baseline: 18.3854x; 18.3854x over previous
import functools

import jax
import jax.numpy as jnp
from jax import lax
from jax.experimental import pallas as pl
from jax.experimental.pallas import tpu as pltpu
from jax.experimental.pallas import tpu_sc as plsc

V = 100000
NS = 64
DM = 1024
HIST = 200
BLK = 2048
NBLK = 49
CAP = 4096
TOKP = 256
NEG = -1e30

_OFFS = tuple([0.25 * (j + 1) for j in range(16)] +
              [4.5, 5.0, 6.0, 7.0, 8.0, 10.0, 12.0, 16.0,
               20.0, 28.0, 40.0, 56.0, 80.0, 112.0, 160.0, 224.0])
_MAXOFF = 224.0
_TARGET = 1224.0


def _k1_body(h_ref, e_ref, tok_ref, logits_ref, rowmax_ref, counts_ref):
    pid = pl.program_id(0)
    x = jax.lax.dot_general(h_ref[...], e_ref[...],
                            (((1,), (1,)), ((), ())),
                            preferred_element_type=jnp.float32)
    col = pid * BLK + jax.lax.broadcasted_iota(jnp.int32, (NS, BLK), 1)
    x = jnp.where(col < V, x, NEG)
    logits_ref[...] = x
    m = jnp.max(x, axis=1, keepdims=True)
    mb = jnp.broadcast_to(m, (NS, 128))

    @pl.when(pid == 0)
    def _():
        rowmax_ref[...] = mb
        tok = tok_ref[...]
        cnt = jnp.zeros((NS, TOKP), jnp.float32)
        for j in range(HIST):
            cnt = cnt + (tok == tok[:, j:j + 1]).astype(jnp.float32)
        counts_ref[...] = cnt

    @pl.when(pid > 0)
    def _():
        rowmax_ref[...] = jnp.maximum(rowmax_ref[...], mb)


def _k1(hidden, emb, tok_p):
    return pl.pallas_call(
        _k1_body,
        grid=(NBLK,),
        in_specs=[
            pl.BlockSpec((NS, DM), lambda i: (0, 0)),
            pl.BlockSpec((BLK, DM), lambda i: (i, 0)),
            pl.BlockSpec((NS, TOKP), lambda i: (0, 0)),
        ],
        out_specs=[
            pl.BlockSpec((NS, BLK), lambda i: (0, i)),
            pl.BlockSpec((NS, 128), lambda i: (0, 0)),
            pl.BlockSpec((NS, TOKP), lambda i: (0, 0)),
        ],
        out_shape=[
            jax.ShapeDtypeStruct((NS, V), jnp.float32),
            jax.ShapeDtypeStruct((NS, 128), jnp.float32),
            jax.ShapeDtypeStruct((NS, TOKP), jnp.float32),
        ],
    )(hidden, emb, tok_p)


def _k2_body(lg_ref, m_ref, t_ref, hist_ref):
    pid = pl.program_id(0)

    @pl.when(pid == 0)
    def _():
        hist_ref[...] = jnp.zeros_like(hist_ref)

    x = lg_ref[...]
    col = pid * BLK + jax.lax.broadcasted_iota(jnp.int32, (NS, BLK), 1)
    x = jnp.where(col < V, x, NEG)
    m0 = m_ref[:, 0:1]
    for j, off in enumerate(_OFFS):
        c = jnp.sum((x > (m0 - off)).astype(jnp.float32), axis=1,
                    keepdims=True)
        hist_ref[:, j:j + 1] += c

    @pl.when(pid == NBLK - 1)
    def _():
        sel = jnp.full((NS, 1), _MAXOFF, jnp.float32)
        for j, off in enumerate(_OFFS):
            ok = hist_ref[:, j:j + 1] >= _TARGET
            sel = jnp.minimum(sel, jnp.where(ok, off, _MAXOFF))
        t_ref[...] = jnp.broadcast_to(m0 - sel, (NS, 128))


def _k2(logits, rowmax):
    return pl.pallas_call(
        _k2_body,
        grid=(NBLK,),
        in_specs=[
            pl.BlockSpec((NS, BLK), lambda i: (0, i)),
            pl.BlockSpec((NS, 128), lambda i: (0, 0)),
        ],
        out_specs=pl.BlockSpec((NS, 128), lambda i: (0, 0)),
        out_shape=jax.ShapeDtypeStruct((NS, 128), jnp.float32),
        scratch_shapes=[pltpu.VMEM((NS, 128), jnp.float32)],
    )(logits, rowmax)


def _lane(vec, k):
    return jnp.sum(jnp.where(lax.iota(jnp.int32, 16) == k, vec, 0.0))


def _k3_body(lg_hbm, tok_hbm, cnt_hbm, par_hbm, t_hbm,
             cv_hbm, ci_hbm,
             row_v, tok_v, cnt_v, par_v, t_v, cv_v, ci_v):
    wid = lax.axis_index("s") * 2 + lax.axis_index("c")
    for dr in range(2):
        r = wid * 2 + dr
        pltpu.sync_copy(lg_hbm.at[r], row_v)
        pltpu.sync_copy(tok_hbm.at[r], tok_v)
        pltpu.sync_copy(cnt_hbm.at[r], cnt_v)
        pltpu.sync_copy(par_hbm.at[r], par_v)
        pltpu.sync_copy(t_hbm.at[r], t_v)
        pv = par_v[pl.ds(0, 16)]
        freq = _lane(pv, 4)
        pres = _lane(pv, 5)
        aneg = _lane(pv, 6)
        apos = _lane(pv, 7)
        tthr = _lane(t_v[pl.ds(0, 16)], 0)
        for i in range(TOKP // 16):
            base = lax.iota(jnp.int32, 16) + i * 16
            msk = base < HIST
            idx = jnp.where(msk, tok_v[pl.ds(i * 16, 16)], 0)
            c = cnt_v[pl.ds(i * 16, 16)]
            x = plsc.load_gather(row_v, [idx], mask=msk)
            y0 = x - freq * c - pres
            y = jnp.where(y0 > 0, y0 + y0 * apos,
                          jnp.where(y0 < 0, y0 + y0 * aneg, y0))
            plsc.store_scatter(row_v, [idx], y, mask=msk)

        negs = jnp.full((16,), NEG, jnp.float32)
        pads = jnp.full((16,), V, jnp.int32)

        def mset(i, _):
            cv_v[pl.ds(i * 16, 16)] = negs
            ci_v[pl.ds(i * 16, 16)] = pads
            return 0

        lax.fori_loop(0, CAP // 16, mset, 0)

        tvec = jnp.full((16,), tthr, jnp.float32)

        def cbody(i, off):
            x = row_v[pl.ds(i * 16, 16)]
            m = x > tvec
            mi = m.astype(jnp.int32)
            pos = off + plsc.cumsum(mi) - 1
            ok = jnp.logical_and(m, pos < CAP)
            posc = jnp.where(ok, pos, 0)
            plsc.store_scatter(cv_v, [posc], x, mask=ok)
            plsc.store_scatter(ci_v, [posc],
                               lax.iota(jnp.int32, 16) + i * 16, mask=ok)
            return off + jnp.sum(mi)

        lax.fori_loop(0, V // 16, cbody, jnp.zeros((16,), jnp.int32))
        pltpu.sync_copy(cv_v, cv_hbm.at[r])
        pltpu.sync_copy(ci_v, ci_hbm.at[r])


def _k3(logits, tok_p, counts, params, t):
    mesh = plsc.VectorSubcoreMesh(core_axis_name="c", subcore_axis_name="s")
    f = functools.partial(
        pl.kernel, _k3_body, mesh=mesh,
        out_type=[
            jax.ShapeDtypeStruct((NS, CAP), jnp.float32),
            jax.ShapeDtypeStruct((NS, CAP), jnp.int32),
        ],
        scratch_types=[
            pltpu.VMEM((V,), jnp.float32),
            pltpu.VMEM((TOKP,), jnp.int32),
            pltpu.VMEM((TOKP,), jnp.float32),
            pltpu.VMEM((128,), jnp.float32),
            pltpu.VMEM((128,), jnp.float32),
            pltpu.VMEM((CAP,), jnp.float32),
            pltpu.VMEM((CAP,), jnp.int32),
        ],
        compiler_params=pltpu.CompilerParams(needs_layout_passes=False),
    )()
    return f(logits, tok_p, counts, params, t)


def _k4_body(cv_ref, par_ref, t_ref, out_ref):
    x = cv_ref[...]
    T = par_ref[:, 0:1]
    P = par_ref[:, 1:2]
    K = par_ref[:, 2:3]
    A = par_ref[:, 3:4]
    m = jnp.max(x, axis=1, keepdims=True)
    lo0 = jnp.minimum(t_ref[:, 0:1], m)
    hi0 = m + 1.0

    def kstep(_, c):
        lo, hi = c
        mid = 0.5 * (lo + hi)
        cnt = jnp.sum((x >= mid).astype(jnp.float32), axis=1, keepdims=True)
        q = cnt >= K
        return jnp.where(q, mid, lo), jnp.where(q, hi, mid)

    lo, hi = lax.fori_loop(0, 30, kstep, (lo0, hi0))
    cnt_above = jnp.sum((x > hi).astype(jnp.float32), axis=1, keepdims=True)
    band = jnp.logical_and(x >= lo, x <= hi)
    topk = jnp.logical_or(x > hi, jnp.logical_and(band, cnt_above < K))

    E = jnp.exp((x - m) / T)
    Ek = jnp.where(topk, E, 0.0)
    Zk = jnp.sum(Ek, axis=1, keepdims=True)
    keep_a = Ek / Zk >= A / (Zk * Zk)
    Pz = P * Zk

    def pstep(_, c):
        lo2, hi2 = c
        mid = 0.5 * (lo2 + hi2)
        s = jnp.sum(jnp.where(x >= mid, Ek, 0.0), axis=1, keepdims=True)
        q = s > Pz
        return jnp.where(q, mid, lo2), jnp.where(q, hi2, mid)

    lo2, hi2 = lax.fori_loop(0, 30, pstep, (lo0, hi0))
    s_above = jnp.sum(jnp.where(x > hi2, Ek, 0.0), axis=1, keepdims=True)
    band2 = jnp.logical_and(x >= lo2, x <= hi2)
    keep_p = jnp.logical_or(
        x > hi2, jnp.logical_and(band2, s_above + Ek <= Pz))

    kept = jnp.logical_and(topk, jnp.logical_and(keep_a, keep_p))
    kept = jnp.logical_or(kept, x == m)
    Ef = jnp.where(kept, E, 0.0)
    out_ref[...] = Ef / jnp.sum(Ef, axis=1, keepdims=True)


def _k4(cand_vals, params, t):
    return pl.pallas_call(
        _k4_body,
        in_specs=[
            pl.BlockSpec((NS, CAP), lambda: (0, 0)),
            pl.BlockSpec((NS, 128), lambda: (0, 0)),
            pl.BlockSpec((NS, 128), lambda: (0, 0)),
        ],
        out_specs=pl.BlockSpec((NS, CAP), lambda: (0, 0)),
        out_shape=jax.ShapeDtypeStruct((NS, CAP), jnp.float32),
    )(cand_vals, params, t)


def _k5_body(pc_hbm, ci_hbm, out_hbm, row_v, pc_v, ci_v):
    wid = lax.axis_index("s") * 2 + lax.axis_index("c")
    zeros = jnp.zeros((16,), jnp.float32)
    for dr in range(2):
        r = wid * 2 + dr
        pltpu.sync_copy(pc_hbm.at[r], pc_v)
        pltpu.sync_copy(ci_hbm.at[r], ci_v)

        def mset(i, _):
            row_v[pl.ds(i * 16, 16)] = zeros
            return 0

        lax.fori_loop(0, V // 16, mset, 0)

        def sbody(i, _):
            idx = ci_v[pl.ds(i * 16, 16)]
            msk = idx < V
            p = pc_v[pl.ds(i * 16, 16)]
            plsc.store_scatter(row_v, [jnp.where(msk, idx, 0)], p, mask=msk)
            return 0

        lax.fori_loop(0, CAP // 16, sbody, 0)
        pltpu.sync_copy(row_v, out_hbm.at[r])


def _k5(probs_cand, cand_idx):
    mesh = plsc.VectorSubcoreMesh(core_axis_name="c", subcore_axis_name="s")
    f = functools.partial(
        pl.kernel, _k5_body, mesh=mesh,
        out_type=jax.ShapeDtypeStruct((NS, V), jnp.float32),
        scratch_types=[
            pltpu.VMEM((V,), jnp.float32),
            pltpu.VMEM((CAP,), jnp.float32),
            pltpu.VMEM((CAP,), jnp.int32),
        ],
        compiler_params=pltpu.CompilerParams(needs_layout_passes=False),
    )()
    return f(probs_cand, cand_idx)


def kernel(hidden_states, embedding, output_tokens, presence_penalties,
           frequency_penalties, repetition_penalties, temperatures,
           top_ps, top_ks, top_as):
    tok = output_tokens.astype(jnp.int32)
    tok_p = jnp.full((NS, TOKP), -1, jnp.int32).at[:, :HIST].set(tok)
    params = jnp.zeros((NS, 128), jnp.float32)
    params = params.at[:, 0].set(temperatures)
    params = params.at[:, 1].set(top_ps)
    params = params.at[:, 2].set(top_ks.astype(jnp.float32))
    params = params.at[:, 3].set(top_as)
    params = params.at[:, 4].set(frequency_penalties)
    params = params.at[:, 5].set(presence_penalties)
    params = params.at[:, 6].set(repetition_penalties - 1.0)
    params = params.at[:, 7].set(1.0 / repetition_penalties - 1.0)

    logits, rowmax, counts = _k1(hidden_states, embedding, tok_p)
    t = _k2(logits, rowmax)
    cand_vals, cand_idx = _k3(logits, tok_p, counts, params, t)
    probs_cand = _k4(cand_vals, params, t)
    return _k5(probs_cand, cand_idx)


# speedup vs baseline: 19.9127x; 1.0831x over previous
import functools

import jax
import jax.numpy as jnp
from jax import lax
from jax.experimental import pallas as pl
from jax.experimental.pallas import tpu as pltpu
from jax.experimental.pallas import tpu_sc as plsc

V = 100000
NS = 64
DM = 1024
HIST = 200
BLK = 2048
NBLK = 49
CAP = 4096
TOKP = 256
NEG = -1e30

_OFFS = tuple([0.25 * (j + 1) for j in range(16)] +
              [4.5, 5.0, 6.0, 7.0, 8.0, 10.0, 12.0, 16.0,
               20.0, 28.0, 40.0, 56.0, 80.0, 112.0, 160.0, 224.0])
_MAXOFF = 224.0
_TARGET = 1224.0


def _k1_body(h_ref, e_ref, tok_ref, logits_ref, rowmax_ref, counts_ref):
    pid = pl.program_id(0)
    x = jax.lax.dot_general(h_ref[...], e_ref[...],
                            (((1,), (1,)), ((), ())),
                            preferred_element_type=jnp.float32)
    col = pid * BLK + jax.lax.broadcasted_iota(jnp.int32, (NS, BLK), 1)
    x = jnp.where(col < V, x, NEG)
    logits_ref[...] = x
    m = jnp.max(x, axis=1, keepdims=True)
    mb = jnp.broadcast_to(m, (NS, 128))

    @pl.when(pid == 0)
    def _():
        rowmax_ref[...] = mb
        tok = tok_ref[...]
        cnt = jnp.zeros((NS, TOKP), jnp.float32)
        for j in range(HIST):
            cnt = cnt + (tok == tok[:, j:j + 1]).astype(jnp.float32)
        counts_ref[...] = cnt

    @pl.when(pid > 0)
    def _():
        rowmax_ref[...] = jnp.maximum(rowmax_ref[...], mb)


def _k1(hidden, emb, tok_p):
    return pl.pallas_call(
        _k1_body,
        grid=(NBLK,),
        in_specs=[
            pl.BlockSpec((NS, DM), lambda i: (0, 0)),
            pl.BlockSpec((BLK, DM), lambda i: (i, 0)),
            pl.BlockSpec((NS, TOKP), lambda i: (0, 0)),
        ],
        out_specs=[
            pl.BlockSpec((NS, BLK), lambda i: (0, i)),
            pl.BlockSpec((NS, 128), lambda i: (0, 0)),
            pl.BlockSpec((NS, TOKP), lambda i: (0, 0)),
        ],
        out_shape=[
            jax.ShapeDtypeStruct((NS, V), jnp.float32),
            jax.ShapeDtypeStruct((NS, 128), jnp.float32),
            jax.ShapeDtypeStruct((NS, TOKP), jnp.float32),
        ],
    )(hidden, emb, tok_p)


def _k2_body(lg_ref, m_ref, t_ref, hist_ref):
    pid = pl.program_id(0)

    @pl.when(pid == 0)
    def _():
        hist_ref[...] = jnp.zeros_like(hist_ref)

    x = lg_ref[...]
    col = pid * BLK + jax.lax.broadcasted_iota(jnp.int32, (NS, BLK), 1)
    x = jnp.where(col < V, x, NEG)
    m0 = m_ref[:, 0:1]
    for j, off in enumerate(_OFFS):
        c = jnp.sum((x > (m0 - off)).astype(jnp.float32), axis=1,
                    keepdims=True)
        hist_ref[:, j:j + 1] += c

    @pl.when(pid == NBLK - 1)
    def _():
        sel = jnp.full((NS, 1), _MAXOFF, jnp.float32)
        for j, off in enumerate(_OFFS):
            ok = hist_ref[:, j:j + 1] >= _TARGET
            sel = jnp.minimum(sel, jnp.where(ok, off, _MAXOFF))
        t_ref[...] = jnp.broadcast_to(m0 - sel, (NS, 128))


def _k2(logits, rowmax):
    return pl.pallas_call(
        _k2_body,
        grid=(NBLK,),
        in_specs=[
            pl.BlockSpec((NS, BLK), lambda i: (0, i)),
            pl.BlockSpec((NS, 128), lambda i: (0, 0)),
        ],
        out_specs=pl.BlockSpec((NS, 128), lambda i: (0, 0)),
        out_shape=jax.ShapeDtypeStruct((NS, 128), jnp.float32),
        scratch_shapes=[pltpu.VMEM((NS, 128), jnp.float32)],
    )(logits, rowmax)


def _lane(vec, k):
    return jnp.sum(jnp.where(lax.iota(jnp.int32, 16) == k, vec, 0.0))


def _k3_body(lg_hbm, tok_hbm, cnt_hbm, par_hbm, t_hbm,
             cv_hbm, ci_hbm,
             row_v, tok_v, cnt_v, par_v, t_v, cv_v, ci_v):
    wid = lax.axis_index("s") * 2 + lax.axis_index("c")
    for dr in range(2):
        r = wid * 2 + dr
        pltpu.sync_copy(lg_hbm.at[r], row_v)
        pltpu.sync_copy(tok_hbm.at[r], tok_v)
        pltpu.sync_copy(cnt_hbm.at[r], cnt_v)
        pltpu.sync_copy(par_hbm.at[r], par_v)
        pltpu.sync_copy(t_hbm.at[r], t_v)
        pv = par_v[pl.ds(0, 16)]
        freq = _lane(pv, 4)
        pres = _lane(pv, 5)
        aneg = _lane(pv, 6)
        apos = _lane(pv, 7)
        tthr = _lane(t_v[pl.ds(0, 16)], 0)
        for i in range(TOKP // 16):
            base = lax.iota(jnp.int32, 16) + i * 16
            msk = base < HIST
            idx = jnp.where(msk, tok_v[pl.ds(i * 16, 16)], 0)
            c = cnt_v[pl.ds(i * 16, 16)]
            x = plsc.load_gather(row_v, [idx], mask=msk)
            y0 = x - freq * c - pres
            y = jnp.where(y0 > 0, y0 + y0 * apos,
                          jnp.where(y0 < 0, y0 + y0 * aneg, y0))
            plsc.store_scatter(row_v, [idx], y, mask=msk)

        negs = jnp.full((16,), NEG, jnp.float32)
        pads = jnp.full((16,), V, jnp.int32)

        def mset(i, _):
            cv_v[pl.ds(i * 16, 16)] = negs
            ci_v[pl.ds(i * 16, 16)] = pads
            return 0

        lax.fori_loop(0, CAP // 16, mset, 0)

        tvec = jnp.full((16,), tthr, jnp.float32)

        def cbody(i, off):
            for u in range(5):
                base = (i * 5 + u) * 16
                x = row_v[pl.ds(base, 16)]
                m = x > tvec
                mi = m.astype(jnp.int32)
                pos = off + plsc.cumsum(mi) - 1
                ok = jnp.logical_and(m, pos < CAP)
                posc = jnp.where(ok, pos, 0)
                plsc.store_scatter(cv_v, [posc], x, mask=ok)
                plsc.store_scatter(ci_v, [posc],
                                   lax.iota(jnp.int32, 16) + base, mask=ok)
                off = off + jnp.sum(mi)
            return off

        lax.fori_loop(0, V // 80, cbody, jnp.zeros((16,), jnp.int32))
        pltpu.sync_copy(cv_v, cv_hbm.at[r])
        pltpu.sync_copy(ci_v, ci_hbm.at[r])


def _k3(logits, tok_p, counts, params, t):
    mesh = plsc.VectorSubcoreMesh(core_axis_name="c", subcore_axis_name="s")
    f = functools.partial(
        pl.kernel, _k3_body, mesh=mesh,
        out_type=[
            jax.ShapeDtypeStruct((NS, CAP), jnp.float32),
            jax.ShapeDtypeStruct((NS, CAP), jnp.int32),
        ],
        scratch_types=[
            pltpu.VMEM((V,), jnp.float32),
            pltpu.VMEM((TOKP,), jnp.int32),
            pltpu.VMEM((TOKP,), jnp.float32),
            pltpu.VMEM((128,), jnp.float32),
            pltpu.VMEM((128,), jnp.float32),
            pltpu.VMEM((CAP,), jnp.float32),
            pltpu.VMEM((CAP,), jnp.int32),
        ],
        compiler_params=pltpu.CompilerParams(needs_layout_passes=False),
    )()
    return f(logits, tok_p, counts, params, t)


def _k4_body(cv_ref, par_ref, t_ref, out_ref):
    x = cv_ref[...]
    T = par_ref[:, 0:1]
    P = par_ref[:, 1:2]
    K = par_ref[:, 2:3]
    A = par_ref[:, 3:4]
    m = jnp.max(x, axis=1, keepdims=True)
    lo0 = jnp.minimum(t_ref[:, 0:1], m)
    hi0 = m + 1.0

    lo, hi = lo0, hi0
    for _ in range(30):
        mid = 0.5 * (lo + hi)
        cnt = jnp.sum((x >= mid).astype(jnp.float32), axis=1, keepdims=True)
        q = cnt >= K
        lo, hi = jnp.where(q, mid, lo), jnp.where(q, hi, mid)
    cnt_above = jnp.sum((x > hi).astype(jnp.float32), axis=1, keepdims=True)
    band = jnp.logical_and(x >= lo, x <= hi)
    topk = jnp.logical_or(x > hi, jnp.logical_and(band, cnt_above < K))

    E = jnp.exp((x - m) / T)
    Ek = jnp.where(topk, E, 0.0)
    Zk = jnp.sum(Ek, axis=1, keepdims=True)
    keep_a = Ek / Zk >= A / (Zk * Zk)
    Pz = P * Zk

    lo2, hi2 = lo0, hi0
    for _ in range(30):
        mid = 0.5 * (lo2 + hi2)
        sm = jnp.sum(jnp.where(x >= mid, Ek, 0.0), axis=1, keepdims=True)
        q = sm > Pz
        lo2, hi2 = jnp.where(q, mid, lo2), jnp.where(q, hi2, mid)
    s_above = jnp.sum(jnp.where(x > hi2, Ek, 0.0), axis=1, keepdims=True)
    band2 = jnp.logical_and(x >= lo2, x <= hi2)
    keep_p = jnp.logical_or(
        x > hi2, jnp.logical_and(band2, s_above + Ek <= Pz))

    kept = jnp.logical_and(topk, jnp.logical_and(keep_a, keep_p))
    kept = jnp.logical_or(kept, x == m)
    Ef = jnp.where(kept, E, 0.0)
    out_ref[...] = Ef / jnp.sum(Ef, axis=1, keepdims=True)


def _k4(cand_vals, params, t):
    return pl.pallas_call(
        _k4_body,
        in_specs=[
            pl.BlockSpec((NS, CAP), lambda: (0, 0)),
            pl.BlockSpec((NS, 128), lambda: (0, 0)),
            pl.BlockSpec((NS, 128), lambda: (0, 0)),
        ],
        out_specs=pl.BlockSpec((NS, CAP), lambda: (0, 0)),
        out_shape=jax.ShapeDtypeStruct((NS, CAP), jnp.float32),
    )(cand_vals, params, t)


def _k5_body(pc_hbm, ci_hbm, out_hbm, row_v, pc_v, ci_v):
    wid = lax.axis_index("s") * 2 + lax.axis_index("c")
    zeros = jnp.zeros((16,), jnp.float32)
    for dr in range(2):
        r = wid * 2 + dr
        pltpu.sync_copy(pc_hbm.at[r], pc_v)
        pltpu.sync_copy(ci_hbm.at[r], ci_v)

        def mset(i, _):
            for u in range(5):
                row_v[pl.ds((i * 5 + u) * 16, 16)] = zeros
            return 0

        lax.fori_loop(0, V // 80, mset, 0)

        def sbody(i, _):
            idx = ci_v[pl.ds(i * 16, 16)]
            msk = idx < V
            p = pc_v[pl.ds(i * 16, 16)]
            plsc.store_scatter(row_v, [jnp.where(msk, idx, 0)], p, mask=msk)
            return 0

        lax.fori_loop(0, CAP // 16, sbody, 0)
        pltpu.sync_copy(row_v, out_hbm.at[r])


def _k5(probs_cand, cand_idx):
    mesh = plsc.VectorSubcoreMesh(core_axis_name="c", subcore_axis_name="s")
    f = functools.partial(
        pl.kernel, _k5_body, mesh=mesh,
        out_type=jax.ShapeDtypeStruct((NS, V), jnp.float32),
        scratch_types=[
            pltpu.VMEM((V,), jnp.float32),
            pltpu.VMEM((CAP,), jnp.float32),
            pltpu.VMEM((CAP,), jnp.int32),
        ],
        compiler_params=pltpu.CompilerParams(needs_layout_passes=False),
    )()
    return f(probs_cand, cand_idx)


def kernel(hidden_states, embedding, output_tokens, presence_penalties,
           frequency_penalties, repetition_penalties, temperatures,
           top_ps, top_ks, top_as):
    tok = output_tokens.astype(jnp.int32)
    tok_p = jnp.full((NS, TOKP), -1, jnp.int32).at[:, :HIST].set(tok)
    params = jnp.zeros((NS, 128), jnp.float32)
    params = params.at[:, 0].set(temperatures)
    params = params.at[:, 1].set(top_ps)
    params = params.at[:, 2].set(top_ks.astype(jnp.float32))
    params = params.at[:, 3].set(top_as)
    params = params.at[:, 4].set(frequency_penalties)
    params = params.at[:, 5].set(presence_penalties)
    params = params.at[:, 6].set(repetition_penalties - 1.0)
    params = params.at[:, 7].set(1.0 / repetition_penalties - 1.0)

    logits, rowmax, counts = _k1(hidden_states, embedding, tok_p)
    t = _k2(logits, rowmax)
    cand_vals, cand_idx = _k3(logits, tok_p, counts, params, t)
    probs_cand = _k4(cand_vals, params, t)
    return _k5(probs_cand, cand_idx)


# speedup vs baseline: 23.2753x; 1.1689x over previous
import functools

import jax
import jax.numpy as jnp
from jax import lax
from jax.experimental import pallas as pl
from jax.experimental.pallas import tpu as pltpu
from jax.experimental.pallas import tpu_sc as plsc

V = 100000
NS = 64
DM = 1024
HIST = 200
BLK = 4096
NBLK = 25
CAP = 4096
TOKP = 256
NEG = -1e30

_OFFS = tuple([0.25 * (j + 1) for j in range(16)] +
              [4.5, 5.0, 6.0, 7.0, 8.0, 10.0, 12.0, 16.0,
               20.0, 28.0, 40.0, 56.0, 80.0, 112.0, 160.0, 224.0])
_MAXOFF = 224.0
_TARGET = 1224.0


def _k1_body(h_ref, e_ref, tok_ref, logits_ref, rowmax_ref, counts_ref):
    pid = pl.program_id(0)
    x = jax.lax.dot_general(h_ref[...], e_ref[...],
                            (((1,), (1,)), ((), ())),
                            preferred_element_type=jnp.float32)
    col = pid * BLK + jax.lax.broadcasted_iota(jnp.int32, (NS, BLK), 1)
    x = jnp.where(col < V, x, NEG)
    logits_ref[...] = x
    m = jnp.max(x, axis=1, keepdims=True)
    mb = jnp.broadcast_to(m, (NS, 128))

    @pl.when(pid == 0)
    def _():
        rowmax_ref[...] = mb
        tok = tok_ref[...]
        cnt = jnp.zeros((NS, TOKP), jnp.float32)
        for j in range(HIST):
            cnt = cnt + (tok == tok[:, j:j + 1]).astype(jnp.float32)
        counts_ref[...] = cnt

    @pl.when(pid > 0)
    def _():
        rowmax_ref[...] = jnp.maximum(rowmax_ref[...], mb)


def _k1(hidden, emb, tok_p):
    return pl.pallas_call(
        _k1_body,
        grid=(NBLK,),
        in_specs=[
            pl.BlockSpec((NS, DM), lambda i: (0, 0)),
            pl.BlockSpec((BLK, DM), lambda i: (i, 0)),
            pl.BlockSpec((NS, TOKP), lambda i: (0, 0)),
        ],
        out_specs=[
            pl.BlockSpec((NS, BLK), lambda i: (0, i)),
            pl.BlockSpec((NS, 128), lambda i: (0, 0)),
            pl.BlockSpec((NS, TOKP), lambda i: (0, 0)),
        ],
        out_shape=[
            jax.ShapeDtypeStruct((NS, V), jnp.float32),
            jax.ShapeDtypeStruct((NS, 128), jnp.float32),
            jax.ShapeDtypeStruct((NS, TOKP), jnp.float32),
        ],
    )(hidden, emb, tok_p)


def _k2_body(lg_ref, m_ref, t_ref, hist_ref):
    pid = pl.program_id(0)

    @pl.when(pid == 0)
    def _():
        hist_ref[...] = jnp.zeros_like(hist_ref)

    x = lg_ref[...]
    col = pid * BLK + jax.lax.broadcasted_iota(jnp.int32, (NS, BLK), 1)
    x = jnp.where(col < V, x, NEG)
    m0 = m_ref[:, 0:1]
    for j, off in enumerate(_OFFS):
        c = jnp.sum((x > (m0 - off)).astype(jnp.float32), axis=1,
                    keepdims=True)
        hist_ref[:, j:j + 1] += c

    @pl.when(pid == NBLK - 1)
    def _():
        sel = jnp.full((NS, 1), _MAXOFF, jnp.float32)
        for j, off in enumerate(_OFFS):
            ok = hist_ref[:, j:j + 1] >= _TARGET
            sel = jnp.minimum(sel, jnp.where(ok, off, _MAXOFF))
        t_ref[...] = jnp.broadcast_to(m0 - sel, (NS, 128))


def _k2(logits, rowmax):
    return pl.pallas_call(
        _k2_body,
        grid=(NBLK,),
        in_specs=[
            pl.BlockSpec((NS, BLK), lambda i: (0, i)),
            pl.BlockSpec((NS, 128), lambda i: (0, 0)),
        ],
        out_specs=pl.BlockSpec((NS, 128), lambda i: (0, 0)),
        out_shape=jax.ShapeDtypeStruct((NS, 128), jnp.float32),
        scratch_shapes=[pltpu.VMEM((NS, 128), jnp.float32)],
    )(logits, rowmax)


def _lane(vec, k):
    return jnp.sum(jnp.where(lax.iota(jnp.int32, 16) == k, vec, 0.0))


def _k3_body(lg_hbm, tok_hbm, cnt_hbm, par_hbm, t_hbm,
             cv_hbm, ci_hbm,
             row_v, tok_v, cnt_v, par_v, t_v, cv_v, ci_v):
    wid = lax.axis_index("s") * 2 + lax.axis_index("c")
    for dr in range(2):
        r = wid * 2 + dr
        pltpu.sync_copy(lg_hbm.at[r], row_v)
        pltpu.sync_copy(tok_hbm.at[r], tok_v)
        pltpu.sync_copy(cnt_hbm.at[r], cnt_v)
        pltpu.sync_copy(par_hbm.at[r], par_v)
        pltpu.sync_copy(t_hbm.at[r], t_v)
        pv = par_v[pl.ds(0, 16)]
        freq = _lane(pv, 4)
        pres = _lane(pv, 5)
        aneg = _lane(pv, 6)
        apos = _lane(pv, 7)
        tthr = _lane(t_v[pl.ds(0, 16)], 0)
        for i in range(TOKP // 16):
            base = lax.iota(jnp.int32, 16) + i * 16
            msk = base < HIST
            idx = jnp.where(msk, tok_v[pl.ds(i * 16, 16)], 0)
            c = cnt_v[pl.ds(i * 16, 16)]
            x = plsc.load_gather(row_v, [idx], mask=msk)
            y0 = x - freq * c - pres
            y = jnp.where(y0 > 0, y0 + y0 * apos,
                          jnp.where(y0 < 0, y0 + y0 * aneg, y0))
            plsc.store_scatter(row_v, [idx], y, mask=msk)

        negs = jnp.full((16,), NEG, jnp.float32)
        pads = jnp.full((16,), V, jnp.int32)

        def mset(i, _):
            cv_v[pl.ds(i * 16, 16)] = negs
            ci_v[pl.ds(i * 16, 16)] = pads
            return 0

        lax.fori_loop(0, CAP // 16, mset, 0)

        tvec = jnp.full((16,), tthr, jnp.float32)

        def cbody(i, off):
            for u in range(5):
                base = (i * 5 + u) * 16
                x = row_v[pl.ds(base, 16)]
                m = x > tvec
                mi = m.astype(jnp.int32)
                pos = off + plsc.cumsum(mi) - 1
                ok = jnp.logical_and(m, pos < CAP)
                posc = jnp.where(ok, pos, 0)
                plsc.store_scatter(cv_v, [posc], x, mask=ok)
                plsc.store_scatter(ci_v, [posc],
                                   lax.iota(jnp.int32, 16) + base, mask=ok)
                off = off + jnp.sum(mi)
            return off

        lax.fori_loop(0, V // 80, cbody, jnp.zeros((16,), jnp.int32))
        pltpu.sync_copy(cv_v, cv_hbm.at[r])
        pltpu.sync_copy(ci_v, ci_hbm.at[r])


def _k3(logits, tok_p, counts, params, t):
    mesh = plsc.VectorSubcoreMesh(core_axis_name="c", subcore_axis_name="s")
    f = functools.partial(
        pl.kernel, _k3_body, mesh=mesh,
        out_type=[
            jax.ShapeDtypeStruct((NS, CAP), jnp.float32),
            jax.ShapeDtypeStruct((NS, CAP), jnp.int32),
        ],
        scratch_types=[
            pltpu.VMEM((V,), jnp.float32),
            pltpu.VMEM((TOKP,), jnp.int32),
            pltpu.VMEM((TOKP,), jnp.float32),
            pltpu.VMEM((128,), jnp.float32),
            pltpu.VMEM((128,), jnp.float32),
            pltpu.VMEM((CAP,), jnp.float32),
            pltpu.VMEM((CAP,), jnp.int32),
        ],
        compiler_params=pltpu.CompilerParams(needs_layout_passes=False),
    )()
    return f(logits, tok_p, counts, params, t)


def _k4_body(cv_ref, par_ref, t_ref, out_ref):
    x = cv_ref[...]
    T = par_ref[:, 0:1]
    P = par_ref[:, 1:2]
    K = par_ref[:, 2:3]
    A = par_ref[:, 3:4]
    m = jnp.max(x, axis=1, keepdims=True)
    lo0 = jnp.minimum(t_ref[:, 0:1], m)
    hi0 = m + 1.0

    lo, hi = lo0, hi0
    for _ in range(30):
        mid = 0.5 * (lo + hi)
        cnt = jnp.sum((x >= mid).astype(jnp.float32), axis=1, keepdims=True)
        q = cnt >= K
        lo, hi = jnp.where(q, mid, lo), jnp.where(q, hi, mid)
    cnt_above = jnp.sum((x > hi).astype(jnp.float32), axis=1, keepdims=True)
    band = jnp.logical_and(x >= lo, x <= hi)
    topk = jnp.logical_or(x > hi, jnp.logical_and(band, cnt_above < K))

    E = jnp.exp((x - m) / T)
    Ek = jnp.where(topk, E, 0.0)
    Zk = jnp.sum(Ek, axis=1, keepdims=True)
    keep_a = Ek / Zk >= A / (Zk * Zk)
    Pz = P * Zk

    lo2, hi2 = lo0, hi0
    for _ in range(30):
        mid = 0.5 * (lo2 + hi2)
        sm = jnp.sum(jnp.where(x >= mid, Ek, 0.0), axis=1, keepdims=True)
        q = sm > Pz
        lo2, hi2 = jnp.where(q, mid, lo2), jnp.where(q, hi2, mid)
    s_above = jnp.sum(jnp.where(x > hi2, Ek, 0.0), axis=1, keepdims=True)
    band2 = jnp.logical_and(x >= lo2, x <= hi2)
    keep_p = jnp.logical_or(
        x > hi2, jnp.logical_and(band2, s_above + Ek <= Pz))

    kept = jnp.logical_and(topk, jnp.logical_and(keep_a, keep_p))
    kept = jnp.logical_or(kept, x == m)
    Ef = jnp.where(kept, E, 0.0)
    out_ref[...] = Ef / jnp.sum(Ef, axis=1, keepdims=True)


def _k4(cand_vals, params, t):
    return pl.pallas_call(
        _k4_body,
        in_specs=[
            pl.BlockSpec((NS, CAP), lambda: (0, 0)),
            pl.BlockSpec((NS, 128), lambda: (0, 0)),
            pl.BlockSpec((NS, 128), lambda: (0, 0)),
        ],
        out_specs=pl.BlockSpec((NS, CAP), lambda: (0, 0)),
        out_shape=jax.ShapeDtypeStruct((NS, CAP), jnp.float32),
    )(cand_vals, params, t)


def _k5_body(pc_hbm, ci_hbm, out_hbm, row_v, pc_v, ci_v):
    wid = lax.axis_index("s") * 2 + lax.axis_index("c")
    zeros = jnp.zeros((16,), jnp.float32)
    for dr in range(2):
        r = wid * 2 + dr
        pltpu.sync_copy(pc_hbm.at[r], pc_v)
        pltpu.sync_copy(ci_hbm.at[r], ci_v)

        def mset(i, _):
            for u in range(5):
                row_v[pl.ds((i * 5 + u) * 16, 16)] = zeros
            return 0

        lax.fori_loop(0, V // 80, mset, 0)

        def sbody(i, _):
            idx = ci_v[pl.ds(i * 16, 16)]
            msk = idx < V
            p = pc_v[pl.ds(i * 16, 16)]
            plsc.store_scatter(row_v, [jnp.where(msk, idx, 0)], p, mask=msk)
            return 0

        lax.fori_loop(0, CAP // 16, sbody, 0)
        pltpu.sync_copy(row_v, out_hbm.at[r])


def _k5(probs_cand, cand_idx):
    mesh = plsc.VectorSubcoreMesh(core_axis_name="c", subcore_axis_name="s")
    f = functools.partial(
        pl.kernel, _k5_body, mesh=mesh,
        out_type=jax.ShapeDtypeStruct((NS, V), jnp.float32),
        scratch_types=[
            pltpu.VMEM((V,), jnp.float32),
            pltpu.VMEM((CAP,), jnp.float32),
            pltpu.VMEM((CAP,), jnp.int32),
        ],
        compiler_params=pltpu.CompilerParams(needs_layout_passes=False),
    )()
    return f(probs_cand, cand_idx)


def kernel(hidden_states, embedding, output_tokens, presence_penalties,
           frequency_penalties, repetition_penalties, temperatures,
           top_ps, top_ks, top_as):
    tok = output_tokens.astype(jnp.int32)
    tok_p = jnp.full((NS, TOKP), -1, jnp.int32).at[:, :HIST].set(tok)
    params = jnp.zeros((NS, 128), jnp.float32)
    params = params.at[:, 0].set(temperatures)
    params = params.at[:, 1].set(top_ps)
    params = params.at[:, 2].set(top_ks.astype(jnp.float32))
    params = params.at[:, 3].set(top_as)
    params = params.at[:, 4].set(frequency_penalties)
    params = params.at[:, 5].set(presence_penalties)
    params = params.at[:, 6].set(repetition_penalties - 1.0)
    params = params.at[:, 7].set(1.0 / repetition_penalties - 1.0)

    logits, rowmax, counts = _k1(hidden_states, embedding, tok_p)
    t = _k2(logits, rowmax)
    cand_vals, cand_idx = _k3(logits, tok_p, counts, params, t)
    probs_cand = _k4(cand_vals, params, t)
    return _k5(probs_cand, cand_idx)
